# baseline (speedup 1.0000x reference)
import jax
import jax.numpy as jnp
from jax import lax
from jax.experimental import pallas as pl
from jax.experimental.pallas import tpu as pltpu
from jax.experimental.pallas import tpu_sc as plsc

NC = 2
NS = 16
NW = NC * NS
L = 16
NSLOT = 3

NPAIR = 320000
CH = 128
NCHUNK = 2 * NPAIR // CH
PBOUND = NPAIR // CH
KTOT = (NCHUNK + NW - 1) // NW

THRESH = 2.5
D = 128


def _twin_body(idx0, idx1, xT, xS, out,
               ixA0, ixA1, ixA2, ixB0, ixB1, ixB2,
               A0, A1, A2, B0, B1, B2, cnt_v,
               sI0, sI1, sI2, sA0, sA1, sA2,
               sB0, sB1, sB2):
    cid = lax.axis_index("c")
    sid = lax.axis_index("s")
    w = sid * NC + cid
    lane = lax.iota(jnp.int32, L)
    rows = [lane + jnp.int32(g * L) for g in range(CH // L)]
    ixA = (ixA0, ixA1, ixA2)
    ixB = (ixB0, ixB1, ixB2)
    Abuf = (A0, A1, A2)
    Bbuf = (B0, B1, B2)
    semI = (sI0, sI1, sI2)
    semA = (sA0, sA1, sA2)
    semB = (sB0, sB1, sB2)

    def idx_issue(k, slot):
        c = jnp.minimum(w + k * jnp.int32(NW), jnp.int32(NCHUNK - 1))
        base = c * jnp.int32(CH)
        pltpu.async_copy(idx0.at[pl.ds(base, CH)], ixA[slot], semI[slot])
        pltpu.async_copy(idx1.at[pl.ds(base, CH)], ixB[slot], semI[slot])

    def idx_wait(slot):
        pltpu.make_async_copy(idx0.at[pl.ds(0, CH)], ixA[slot],
                              semI[slot]).wait()
        pltpu.make_async_copy(idx1.at[pl.ds(0, CH)], ixB[slot],
                              semI[slot]).wait()

    def issue(slot):
        pltpu.async_copy(xT.at[ixA[slot]], Abuf[slot], semA[slot])
        pltpu.async_copy(xS.at[ixB[slot]], Bbuf[slot], semB[slot])

    def wait_slot(slot):
        pltpu.make_async_copy(xT.at[ixA[slot]], Abuf[slot],
                              semA[slot]).wait()
        pltpu.make_async_copy(xS.at[ixB[slot]], Bbuf[slot],
                              semB[slot]).wait()

    def compute(k, slot, cntP, cntN):
        c = w + k * jnp.int32(NW)
        act = (c < jnp.int32(NCHUNK)).astype(jnp.int32)
        isp = (c < jnp.int32(PBOUND)).astype(jnp.int32)
        rp = jnp.full((L,), act * isp, dtype=jnp.int32)
        rn = jnp.full((L,), act * (1 - isp), dtype=jnp.int32)
        A = Abuf[slot]
        B = Bbuf[slot]

        def dstep(d, accs):
            col = (lane + d) & jnp.int32(D - 1)
            new = []
            for g in range(CH // L):
                va = plsc.load_gather(A, [rows[g], col])
                vb = plsc.load_gather(B, [rows[g], col])
                t = va - vb
                new.append(accs[g] + t * t)
            return tuple(new)

        zf = jnp.zeros((L,), jnp.float32)
        accs = lax.fori_loop(
            jnp.int32(0), jnp.int32(D), dstep,
            tuple(zf for _ in range(CH // L)))
        for g in range(CH // L):
            cntP = cntP + (accs[g] > THRESH).astype(jnp.int32) * rp
            cntN = cntN + (accs[g] < THRESH).astype(jnp.int32) * rn
        return cntP, cntN

    for s in range(NSLOT - 1):
        idx_issue(jnp.int32(s), s)
    for s in range(NSLOT - 1):
        idx_wait(s)
        issue(s)
    idx_issue(jnp.int32(NSLOT - 1), NSLOT - 1)

    def quad(kk, carry):
        cntP, cntN = carry
        k0 = kk * jnp.int32(NSLOT)
        for s in range(NSLOT):
            k = k0 + jnp.int32(s)
            wait_slot(s)
            idx_issue(k + jnp.int32(NSLOT), s)
            idx_wait((s + NSLOT - 1) % NSLOT)
            issue((s + NSLOT - 1) % NSLOT)
            cntP, cntN = compute(k, s, cntP, cntN)
        return cntP, cntN

    zero = jnp.zeros((L,), jnp.int32)
    cntP, cntN = lax.fori_loop(jnp.int32(0), jnp.int32((KTOT - 1) // NSLOT),
                               quad, (zero, zero))
    KQ = ((KTOT - 1) // NSLOT) * NSLOT
    for k in range(KQ, KTOT):
        wait_slot(k % NSLOT)
        cntP, cntN = compute(jnp.int32(k), k % NSLOT, cntP, cntN)
    for g in range(KTOT, KQ + NSLOT - 1):
        wait_slot(g % NSLOT)
    idx_wait((KQ + NSLOT - 1) % NSLOT)

    cnt_v[...] = cntP
    pltpu.sync_copy(cnt_v, out.at[jnp.int32(0), w])
    cnt_v[...] = cntN
    pltpu.sync_copy(cnt_v, out.at[jnp.int32(1), w])


@jax.jit
def _twin_counts(idx0, idx1, xT, xS):
    mesh = plsc.VectorSubcoreMesh(core_axis_name="c", subcore_axis_name="s")
    return pl.kernel(
        _twin_body,
        out_type=jax.ShapeDtypeStruct((2, NW, L), jnp.int32),
        mesh=mesh,
        scratch_types=(
            [pltpu.VMEM((CH,), jnp.int32) for _ in range(2 * NSLOT)]
            + [pltpu.VMEM((CH, D), jnp.float32) for _ in range(2 * NSLOT)]
            + [pltpu.VMEM((L,), jnp.int32)]
            + [pltpu.SemaphoreType.DMA for _ in range(3 * NSLOT)]
        ),
        compiler_params=pltpu.CompilerParams(needs_layout_passes=False),
    )(idx0, idx1, xT, xS)


def kernel(xS, xT, p_, n_):
    idx0 = jnp.concatenate([p_[:, 0], n_[:, 0]]).astype(jnp.int32)
    idx1 = jnp.concatenate([p_[:, 1], n_[:, 1]]).astype(jnp.int32)
    out = _twin_counts(idx0, idx1, xT, xS)
    nFN = jnp.sum(out[0]).astype(jnp.int64)
    nFP = jnp.sum(out[1]).astype(jnp.int64)
    return (nFN, nFP)


# speedup vs baseline: 1.0164x; 1.0164x over previous
import jax
import jax.numpy as jnp
from jax import lax
from jax.experimental import pallas as pl
from jax.experimental.pallas import tpu as pltpu
from jax.experimental.pallas import tpu_sc as plsc

NC = 2
NS = 16
NW = NC * NS
L = 16
NSLOT = 4

NPAIR = 320000
CH = 80
NCHUNK = 2 * NPAIR // CH
PBOUND = NPAIR // CH
KTOT = (NCHUNK + NW - 1) // NW

THRESH = 2.5
D = 128


def _twin_body(idx0, idx1, xT, xS, out,
               ixA0, ixA1, ixA2, ixA3, ixB0, ixB1, ixB2, ixB3,
               A0, A1, A2, A3, B0, B1, B2, B3, cnt_v,
               sI0, sI1, sI2, sI3, sA0, sA1, sA2, sA3,
               sB0, sB1, sB2, sB3):
    cid = lax.axis_index("c")
    sid = lax.axis_index("s")
    w = sid * NC + cid
    lane = lax.iota(jnp.int32, L)
    rows = [lane + jnp.int32(g * L) for g in range(CH // L)]
    ixA = (ixA0, ixA1, ixA2, ixA3)
    ixB = (ixB0, ixB1, ixB2, ixB3)
    Abuf = (A0, A1, A2, A3)
    Bbuf = (B0, B1, B2, B3)
    semI = (sI0, sI1, sI2, sI3)
    semA = (sA0, sA1, sA2, sA3)
    semB = (sB0, sB1, sB2, sB3)

    def idx_issue(k, slot):
        c = jnp.minimum(w + k * jnp.int32(NW), jnp.int32(NCHUNK - 1))
        base = c * jnp.int32(CH)
        pltpu.async_copy(idx0.at[pl.ds(base, CH)], ixA[slot], semI[slot])
        pltpu.async_copy(idx1.at[pl.ds(base, CH)], ixB[slot], semI[slot])

    def idx_wait(slot):
        pltpu.make_async_copy(idx0.at[pl.ds(0, CH)], ixA[slot],
                              semI[slot]).wait()
        pltpu.make_async_copy(idx1.at[pl.ds(0, CH)], ixB[slot],
                              semI[slot]).wait()

    def issue(slot):
        pltpu.async_copy(xT.at[ixA[slot]], Abuf[slot], semA[slot])
        pltpu.async_copy(xS.at[ixB[slot]], Bbuf[slot], semB[slot])

    def wait_slot(slot):
        pltpu.make_async_copy(xT.at[ixA[slot]], Abuf[slot],
                              semA[slot]).wait()
        pltpu.make_async_copy(xS.at[ixB[slot]], Bbuf[slot],
                              semB[slot]).wait()

    def compute(k, slot, cntP, cntN):
        c = w + k * jnp.int32(NW)
        act = (c < jnp.int32(NCHUNK)).astype(jnp.int32)
        isp = (c < jnp.int32(PBOUND)).astype(jnp.int32)
        rp = jnp.full((L,), act * isp, dtype=jnp.int32)
        rn = jnp.full((L,), act * (1 - isp), dtype=jnp.int32)
        A = Abuf[slot]
        B = Bbuf[slot]

        def dstep(d, accs):
            col = (lane + d) & jnp.int32(D - 1)
            new = []
            for g in range(CH // L):
                va = plsc.load_gather(A, [rows[g], col])
                vb = plsc.load_gather(B, [rows[g], col])
                t = va - vb
                new.append(accs[g] + t * t)
            return tuple(new)

        zf = jnp.zeros((L,), jnp.float32)
        accs = lax.fori_loop(
            jnp.int32(0), jnp.int32(D), dstep,
            tuple(zf for _ in range(CH // L)))
        for g in range(CH // L):
            cntP = cntP + (accs[g] > THRESH).astype(jnp.int32) * rp
            cntN = cntN + (accs[g] < THRESH).astype(jnp.int32) * rn
        return cntP, cntN

    for s in range(NSLOT - 1):
        idx_issue(jnp.int32(s), s)
    for s in range(NSLOT - 1):
        idx_wait(s)
        issue(s)
    idx_issue(jnp.int32(NSLOT - 1), NSLOT - 1)

    def quad(kk, carry):
        cntP, cntN = carry
        k0 = kk * jnp.int32(NSLOT)
        for s in range(NSLOT):
            k = k0 + jnp.int32(s)
            wait_slot(s)
            idx_issue(k + jnp.int32(NSLOT), s)
            idx_wait((s + NSLOT - 1) % NSLOT)
            issue((s + NSLOT - 1) % NSLOT)
            cntP, cntN = compute(k, s, cntP, cntN)
        return cntP, cntN

    zero = jnp.zeros((L,), jnp.int32)
    cntP, cntN = lax.fori_loop(jnp.int32(0), jnp.int32((KTOT - 1) // NSLOT),
                               quad, (zero, zero))
    KQ = ((KTOT - 1) // NSLOT) * NSLOT
    for k in range(KQ, KTOT):
        wait_slot(k % NSLOT)
        cntP, cntN = compute(jnp.int32(k), k % NSLOT, cntP, cntN)
    for g in range(KTOT, KQ + NSLOT - 1):
        wait_slot(g % NSLOT)
    idx_wait((KQ + NSLOT - 1) % NSLOT)

    cnt_v[...] = cntP
    pltpu.sync_copy(cnt_v, out.at[jnp.int32(0), w])
    cnt_v[...] = cntN
    pltpu.sync_copy(cnt_v, out.at[jnp.int32(1), w])


@jax.jit
def _twin_counts(idx0, idx1, xT, xS):
    mesh = plsc.VectorSubcoreMesh(core_axis_name="c", subcore_axis_name="s")
    return pl.kernel(
        _twin_body,
        out_type=jax.ShapeDtypeStruct((2, NW, L), jnp.int32),
        mesh=mesh,
        scratch_types=(
            [pltpu.VMEM((CH,), jnp.int32) for _ in range(2 * NSLOT)]
            + [pltpu.VMEM((CH, D), jnp.float32) for _ in range(2 * NSLOT)]
            + [pltpu.VMEM((L,), jnp.int32)]
            + [pltpu.SemaphoreType.DMA for _ in range(3 * NSLOT)]
        ),
        compiler_params=pltpu.CompilerParams(needs_layout_passes=False),
    )(idx0, idx1, xT, xS)


def kernel(xS, xT, p_, n_):
    idx0 = jnp.concatenate([p_[:, 0], n_[:, 0]]).astype(jnp.int32)
    idx1 = jnp.concatenate([p_[:, 1], n_[:, 1]]).astype(jnp.int32)
    out = _twin_counts(idx0, idx1, xT, xS)
    nFN = jnp.sum(out[0]).astype(jnp.int64)
    nFP = jnp.sum(out[1]).astype(jnp.int64)
    return (nFN, nFP)
